# baseline (speedup 1.0000x reference)
import functools

import jax
import jax.numpy as jnp
from jax.experimental import pallas as pl
from jax.experimental.pallas import tpu as pltpu

D = 1024
H = 16
HD = 64
HID = 2048
E = 8
S = 2048

BQ = 256
BR = 256
HC = 512


def _ln(x, scale, shift):
    mean = jnp.mean(x, axis=-1, keepdims=True)
    xc = x - mean
    var = jnp.mean(xc * xc, axis=-1, keepdims=True)
    return scale * xc * jax.lax.rsqrt(var + 1e-5) + shift


def _ln_qkv_kernel(x_ref, w_ref, scale_ref, shift_ref, qkv_ref):
    h = _ln(x_ref[...], scale_ref[...], shift_ref[...])
    qkv_ref[...] = jnp.dot(h, w_ref[...], preferred_element_type=jnp.float32)


def _attn_kernel(q_ref, k_ref, v_ref, o_ref):
    i = pl.program_id(1)
    q = q_ref[0]
    k = k_ref[0]
    s = jax.lax.dot_general(q, k, (((1,), (1,)), ((), ())),
                            preferred_element_type=jnp.float32)
    row = i * BQ + jax.lax.broadcasted_iota(jnp.int32, (BQ, S), 0)
    col = jax.lax.broadcasted_iota(jnp.int32, (BQ, S), 1)
    s = jnp.where(col > row, -1e30, s) * (1.0 / (HD ** 0.5))
    m = jnp.max(s, axis=-1, keepdims=True)
    p = jnp.exp(s - m)
    p = p / jnp.sum(p, axis=-1, keepdims=True)
    o_ref[0] = jnp.dot(p, v_ref[0], preferred_element_type=jnp.float32)


def _wo_ln_gate_kernel(ctx_ref, wo_ref, bo_ref, x_ref, scale_ref, shift_ref,
                       gw_ref, x2_ref, h2_ref, wf_ref):
    ctx = ctx_ref[...]
    x2 = jnp.dot(ctx, wo_ref[...], preferred_element_type=jnp.float32)
    x2 = x2 + bo_ref[...] + x_ref[...]
    x2_ref[...] = x2
    h2 = _ln(x2, scale_ref[...], shift_ref[...])
    h2_ref[...] = h2
    s = jnp.dot(h2, gw_ref[...], preferred_element_type=jnp.float32)
    lane = jax.lax.broadcasted_iota(jnp.int32, s.shape, 1)
    v1 = jnp.max(s, axis=-1, keepdims=True)
    e1 = jnp.min(jnp.where(s == v1, lane, E), axis=-1, keepdims=True)
    is1 = lane == e1
    s2 = jnp.where(is1, -jnp.inf, s)
    v2 = jnp.max(s2, axis=-1, keepdims=True)
    e2 = jnp.min(jnp.where(s2 == v2, lane, E), axis=-1, keepdims=True)
    is2 = lane == e2
    z = jnp.exp(v2 - v1)
    denom = 1.0 + z
    w = jnp.where(is1, 1.0 / denom, 0.0) + jnp.where(is2, z / denom, 0.0)
    wf_ref[...] = w


BT = 256
NB = (S * 2 + E * (BT - 1) + BT - 1) // BT
NBT = NB * BT
NHC = HID // HC


def _moe_routed_kernel(be_ref, tok_ref, wt_ref, used_ref,
                       h2_ref, x2_ref, fc1_ref, fc2_ref, fc3_ref,
                       out_ref, xg_ref, eo_ref):
    b = pl.program_id(0)
    hc = pl.program_id(1)

    @pl.when((b == 0) & (hc == 0))
    def _():
        out_ref[...] = x2_ref[...]

    @pl.when(used_ref[b] == 1)
    def _():
        base = b * BT

        @pl.when(hc == 0)
        def _():
            def gbody(r, _):
                t = tok_ref[base + r]
                xg_ref[pl.ds(r, 1), :] = h2_ref[pl.ds(t, 1), :]
                return 0
            jax.lax.fori_loop(0, BT, gbody, 0)

        xg = xg_ref[...]
        a = jnp.dot(xg, fc1_ref[0], preferred_element_type=jnp.float32)
        g = jnp.dot(xg, fc2_ref[0], preferred_element_type=jnp.float32)
        hidden = a * jax.lax.logistic(a) * g
        part = jnp.dot(hidden, fc3_ref[0], preferred_element_type=jnp.float32)

        @pl.when(hc == 0)
        def _():
            eo_ref[...] = part

        @pl.when(hc != 0)
        def _():
            eo_ref[...] += part

        @pl.when(hc == NHC - 1)
        def _():
            def sbody(r, _):
                t = tok_ref[base + r]
                w = wt_ref[base + r]
                out_ref[pl.ds(t, 1), :] += w * eo_ref[pl.ds(r, 1), :]
                return 0
            jax.lax.fori_loop(0, BT, sbody, 0)


def _routing_tables(wf):
    m = wf != 0.0
    cnt = jnp.sum(m.astype(jnp.int32), axis=0)
    padded = ((cnt + BT - 1) // BT) * BT
    ends = jnp.cumsum(padded)
    off = ends - padded
    total = ends[-1]
    pos = jnp.cumsum(m.astype(jnp.int32), axis=0) - 1
    slot = off[None, :] + pos
    slot_flat = jnp.where(m, slot, NBT).reshape(-1)
    tokids = jnp.broadcast_to(
        jnp.arange(S, dtype=jnp.int32)[:, None], (S, E)).reshape(-1)
    tok_table = jnp.zeros(NBT + 1, jnp.int32).at[slot_flat].set(tokids)[:NBT]
    wt_table = jnp.zeros(NBT + 1, jnp.float32).at[slot_flat].set(
        wf.reshape(-1))[:NBT]
    sbt = jnp.arange(NB, dtype=jnp.int32) * BT
    be = jnp.sum((sbt[:, None] >= ends[None, :]).astype(jnp.int32), axis=1)
    be = jnp.minimum(be, E - 1)
    used = (sbt < total).astype(jnp.int32)
    nb_used = jnp.maximum((total + BT - 1) // BT, 1)
    be = jnp.where(used == 1, be, jnp.take(be, nb_used - 1))
    return be, tok_table, wt_table, used


def kernel(x, Wq, Wk, Wv, Wo, bo, n1_scale, n1_shift, n2_scale, n2_shift,
           gate_w, fc1_w, fc2_w, fc3_w):
    b, s, d = x.shape
    xf = x.reshape(s, d)
    wqkv = jnp.concatenate([Wq, Wk, Wv], axis=1)
    n1_scale2 = n1_scale.reshape(1, d)
    n1_shift2 = n1_shift.reshape(1, d)
    n2_scale2 = n2_scale.reshape(1, d)
    n2_shift2 = n2_shift.reshape(1, d)
    bo2 = bo.reshape(1, d)

    qkv = pl.pallas_call(
        _ln_qkv_kernel,
        grid=(S // BR,),
        in_specs=[
            pl.BlockSpec((BR, D), lambda i: (i, 0)),
            pl.BlockSpec((D, 3 * D), lambda i: (0, 0)),
            pl.BlockSpec((1, D), lambda i: (0, 0)),
            pl.BlockSpec((1, D), lambda i: (0, 0)),
        ],
        out_specs=pl.BlockSpec((BR, 3 * D), lambda i: (i, 0)),
        out_shape=jax.ShapeDtypeStruct((S, 3 * D), jnp.float32),
    )(xf, wqkv, n1_scale2, n1_shift2)

    qkv3 = qkv.reshape(S, 3 * H, HD).transpose(1, 0, 2)

    ctx3 = pl.pallas_call(
        _attn_kernel,
        grid=(H, S // BQ),
        in_specs=[
            pl.BlockSpec((1, BQ, HD), lambda h, i: (h, i, 0)),
            pl.BlockSpec((1, S, HD), lambda h, i: (H + h, 0, 0)),
            pl.BlockSpec((1, S, HD), lambda h, i: (2 * H + h, 0, 0)),
        ],
        out_specs=pl.BlockSpec((1, BQ, HD), lambda h, i: (h, i, 0)),
        out_shape=jax.ShapeDtypeStruct((H, S, HD), jnp.float32),
    )(qkv3, qkv3, qkv3)
    ctx = ctx3.transpose(1, 0, 2).reshape(S, D)

    x2, h2, wf = pl.pallas_call(
        _wo_ln_gate_kernel,
        grid=(S // BR,),
        in_specs=[
            pl.BlockSpec((BR, D), lambda i: (i, 0)),
            pl.BlockSpec((D, D), lambda i: (0, 0)),
            pl.BlockSpec((1, D), lambda i: (0, 0)),
            pl.BlockSpec((BR, D), lambda i: (i, 0)),
            pl.BlockSpec((1, D), lambda i: (0, 0)),
            pl.BlockSpec((1, D), lambda i: (0, 0)),
            pl.BlockSpec((D, E), lambda i: (0, 0)),
        ],
        out_specs=[
            pl.BlockSpec((BR, D), lambda i: (i, 0)),
            pl.BlockSpec((BR, D), lambda i: (i, 0)),
            pl.BlockSpec((BR, E), lambda i: (i, 0)),
        ],
        out_shape=[
            jax.ShapeDtypeStruct((S, D), jnp.float32),
            jax.ShapeDtypeStruct((S, D), jnp.float32),
            jax.ShapeDtypeStruct((S, E), jnp.float32),
        ],
    )(ctx, Wo, bo2, xf, n2_scale2, n2_shift2, gate_w)

    be, tok_table, wt_table, used = _routing_tables(wf)

    grid_spec = pltpu.PrefetchScalarGridSpec(
        num_scalar_prefetch=4,
        grid=(NB, NHC),
        in_specs=[
            pl.BlockSpec((S, D), lambda bb, hc, be, tok, wt, u: (0, 0)),
            pl.BlockSpec((S, D), lambda bb, hc, be, tok, wt, u: (0, 0)),
            pl.BlockSpec((1, D, HC),
                         lambda bb, hc, be, tok, wt, u: (be[bb], 0, hc)),
            pl.BlockSpec((1, D, HC),
                         lambda bb, hc, be, tok, wt, u: (be[bb], 0, hc)),
            pl.BlockSpec((1, HC, D),
                         lambda bb, hc, be, tok, wt, u: (be[bb], hc, 0)),
        ],
        out_specs=pl.BlockSpec((S, D), lambda bb, hc, be, tok, wt, u: (0, 0)),
        scratch_shapes=[
            pltpu.VMEM((BT, D), jnp.float32),
            pltpu.VMEM((BT, D), jnp.float32),
        ],
    )
    out = pl.pallas_call(
        _moe_routed_kernel,
        grid_spec=grid_spec,
        out_shape=jax.ShapeDtypeStruct((S, D), jnp.float32),
    )(be, tok_table, wt_table, used, h2, x2, fc1_w, fc2_w, fc3_w)

    return out.reshape(b, s, d)


# speedup vs baseline: 1.3681x; 1.3681x over previous
import functools

import jax
import jax.numpy as jnp
from jax.experimental import pallas as pl
from jax.experimental.pallas import tpu as pltpu

D = 1024
H = 16
HD = 64
HID = 2048
E = 8
S = 2048

BQ = 256
BR = 256
HC = 512


def _ln(x, scale, shift):
    mean = jnp.mean(x, axis=-1, keepdims=True)
    xc = x - mean
    var = jnp.mean(xc * xc, axis=-1, keepdims=True)
    return scale * xc * jax.lax.rsqrt(var + 1e-5) + shift


def _bdot(a, b):
    return jax.lax.dot_general(
        a.astype(jnp.bfloat16), b.astype(jnp.bfloat16),
        (((1,), (0,)), ((), ())), preferred_element_type=jnp.float32)


def _ln_qkv_kernel(x_ref, w_ref, scale_ref, shift_ref, qkv_ref):
    h = _ln(x_ref[...], scale_ref[...], shift_ref[...])
    qkv_ref[...] = _bdot(h, w_ref[...])


def _attn_kernel(q_ref, k_ref, v_ref, o_ref):
    i = pl.program_id(1)
    q = q_ref[0]
    k = k_ref[0]
    s = jax.lax.dot_general(q.astype(jnp.bfloat16), k.astype(jnp.bfloat16),
                            (((1,), (1,)), ((), ())),
                            preferred_element_type=jnp.float32)
    row = i * BQ + jax.lax.broadcasted_iota(jnp.int32, (BQ, S), 0)
    col = jax.lax.broadcasted_iota(jnp.int32, (BQ, S), 1)
    s = jnp.where(col > row, -1e30, s) * (1.0 / (HD ** 0.5))
    m = jnp.max(s, axis=-1, keepdims=True)
    p = jnp.exp(s - m)
    p = p / jnp.sum(p, axis=-1, keepdims=True)
    o_ref[0] = _bdot(p, v_ref[0])


def _wo_ln_gate_kernel(ctx_ref, wo_ref, bo_ref, x_ref, scale_ref, shift_ref,
                       gw_ref, x2_ref, h2_ref, wf_ref):
    ctx = ctx_ref[...]
    x2 = _bdot(ctx, wo_ref[...])
    x2 = x2 + bo_ref[...] + x_ref[...]
    x2_ref[...] = x2
    h2 = _ln(x2, scale_ref[...], shift_ref[...])
    h2_ref[...] = h2
    s = jnp.dot(h2, gw_ref[...], preferred_element_type=jnp.float32)
    lane = jax.lax.broadcasted_iota(jnp.int32, s.shape, 1)
    v1 = jnp.max(s, axis=-1, keepdims=True)
    e1 = jnp.min(jnp.where(s == v1, lane, E), axis=-1, keepdims=True)
    is1 = lane == e1
    s2 = jnp.where(is1, -jnp.inf, s)
    v2 = jnp.max(s2, axis=-1, keepdims=True)
    e2 = jnp.min(jnp.where(s2 == v2, lane, E), axis=-1, keepdims=True)
    is2 = lane == e2
    z = jnp.exp(v2 - v1)
    denom = 1.0 + z
    w = jnp.where(is1, 1.0 / denom, 0.0) + jnp.where(is2, z / denom, 0.0)
    wf_ref[...] = w


def _moe_dense_kernel(h2_ref, x2_ref, wf_ref, fc1_ref, fc2_ref, fc3_ref,
                      out_ref):
    e = pl.program_id(0)
    hc = pl.program_id(1)

    @pl.when((e == 0) & (hc == 0))
    def _():
        out_ref[...] = x2_ref[...]

    x = h2_ref[...]
    a = _bdot(x, fc1_ref[0])
    g = _bdot(x, fc2_ref[0])
    hidden = a * jax.lax.logistic(a) * g
    eo = _bdot(hidden, fc3_ref[0])
    wf = wf_ref[...]
    lane = jax.lax.broadcasted_iota(jnp.int32, wf.shape, 1)
    wcol = jnp.sum(jnp.where(lane == e, wf, 0.0), axis=-1, keepdims=True)
    out_ref[...] += wcol * eo


BT = 256
NB = (S * 2 + E * (BT - 1) + BT - 1) // BT
NBT = NB * BT
NHC = HID // HC


def _moe_routed_kernel(be_ref, tok_ref, wt_ref, used_ref,
                       h2_ref, x2_ref, fc1_ref, fc2_ref, fc3_ref,
                       out_ref, xg_ref, eo_ref):
    b = pl.program_id(0)
    hc = pl.program_id(1)

    @pl.when((b == 0) & (hc == 0))
    def _():
        out_ref[...] = x2_ref[...]

    @pl.when(used_ref[b] == 1)
    def _():
        base = b * BT

        @pl.when(hc == 0)
        def _():
            def gbody(r, _):
                t = tok_ref[base + r]
                xg_ref[pl.ds(r, 1), :] = h2_ref[pl.ds(t, 1), :]
                return 0
            jax.lax.fori_loop(0, BT, gbody, 0)

        xg = xg_ref[...]
        a = jnp.dot(xg, fc1_ref[0], preferred_element_type=jnp.float32)
        g = jnp.dot(xg, fc2_ref[0], preferred_element_type=jnp.float32)
        hidden = a * jax.lax.logistic(a) * g
        part = jnp.dot(hidden, fc3_ref[0], preferred_element_type=jnp.float32)

        @pl.when(hc == 0)
        def _():
            eo_ref[...] = part

        @pl.when(hc != 0)
        def _():
            eo_ref[...] += part

        @pl.when(hc == NHC - 1)
        def _():
            def sbody(r, _):
                t = tok_ref[base + r]
                w = wt_ref[base + r]
                out_ref[pl.ds(t, 1), :] += w * eo_ref[pl.ds(r, 1), :]
                return 0
            jax.lax.fori_loop(0, BT, sbody, 0)


def _routing_tables(wf):
    m = wf != 0.0
    cnt = jnp.sum(m.astype(jnp.int32), axis=0)
    padded = ((cnt + BT - 1) // BT) * BT
    ends = jnp.cumsum(padded)
    off = ends - padded
    total = ends[-1]
    pos = jnp.cumsum(m.astype(jnp.int32), axis=0) - 1
    slot = off[None, :] + pos
    slot_flat = jnp.where(m, slot, NBT).reshape(-1)
    tokids = jnp.broadcast_to(
        jnp.arange(S, dtype=jnp.int32)[:, None], (S, E)).reshape(-1)
    tok_table = jnp.zeros(NBT + 1, jnp.int32).at[slot_flat].set(tokids)[:NBT]
    wt_table = jnp.zeros(NBT + 1, jnp.float32).at[slot_flat].set(
        wf.reshape(-1))[:NBT]
    sbt = jnp.arange(NB, dtype=jnp.int32) * BT
    be = jnp.sum((sbt[:, None] >= ends[None, :]).astype(jnp.int32), axis=1)
    be = jnp.minimum(be, E - 1)
    used = (sbt < total).astype(jnp.int32)
    nb_used = jnp.maximum((total + BT - 1) // BT, 1)
    be = jnp.where(used == 1, be, jnp.take(be, nb_used - 1))
    return be, tok_table, wt_table, used


def kernel(x, Wq, Wk, Wv, Wo, bo, n1_scale, n1_shift, n2_scale, n2_shift,
           gate_w, fc1_w, fc2_w, fc3_w):
    b, s, d = x.shape
    xf = x.reshape(s, d)
    wqkv = jnp.concatenate([Wq, Wk, Wv], axis=1)
    n1_scale2 = n1_scale.reshape(1, d)
    n1_shift2 = n1_shift.reshape(1, d)
    n2_scale2 = n2_scale.reshape(1, d)
    n2_shift2 = n2_shift.reshape(1, d)
    bo2 = bo.reshape(1, d)

    qkv = pl.pallas_call(
        _ln_qkv_kernel,
        grid=(S // BR,),
        in_specs=[
            pl.BlockSpec((BR, D), lambda i: (i, 0)),
            pl.BlockSpec((D, 3 * D), lambda i: (0, 0)),
            pl.BlockSpec((1, D), lambda i: (0, 0)),
            pl.BlockSpec((1, D), lambda i: (0, 0)),
        ],
        out_specs=pl.BlockSpec((BR, 3 * D), lambda i: (i, 0)),
        out_shape=jax.ShapeDtypeStruct((S, 3 * D), jnp.float32),
    )(xf, wqkv, n1_scale2, n1_shift2)

    qkv3 = qkv.reshape(S, 3 * H, HD).transpose(1, 0, 2)

    ctx3 = pl.pallas_call(
        _attn_kernel,
        grid=(H, S // BQ),
        in_specs=[
            pl.BlockSpec((1, BQ, HD), lambda h, i: (h, i, 0)),
            pl.BlockSpec((1, S, HD), lambda h, i: (H + h, 0, 0)),
            pl.BlockSpec((1, S, HD), lambda h, i: (2 * H + h, 0, 0)),
        ],
        out_specs=pl.BlockSpec((1, BQ, HD), lambda h, i: (h, i, 0)),
        out_shape=jax.ShapeDtypeStruct((H, S, HD), jnp.float32),
    )(qkv3, qkv3, qkv3)
    ctx = ctx3.transpose(1, 0, 2).reshape(S, D)

    x2, h2, wf = pl.pallas_call(
        _wo_ln_gate_kernel,
        grid=(S // BR,),
        in_specs=[
            pl.BlockSpec((BR, D), lambda i: (i, 0)),
            pl.BlockSpec((D, D), lambda i: (0, 0)),
            pl.BlockSpec((1, D), lambda i: (0, 0)),
            pl.BlockSpec((BR, D), lambda i: (i, 0)),
            pl.BlockSpec((1, D), lambda i: (0, 0)),
            pl.BlockSpec((1, D), lambda i: (0, 0)),
            pl.BlockSpec((D, E), lambda i: (0, 0)),
        ],
        out_specs=[
            pl.BlockSpec((BR, D), lambda i: (i, 0)),
            pl.BlockSpec((BR, D), lambda i: (i, 0)),
            pl.BlockSpec((BR, E), lambda i: (i, 0)),
        ],
        out_shape=[
            jax.ShapeDtypeStruct((S, D), jnp.float32),
            jax.ShapeDtypeStruct((S, D), jnp.float32),
            jax.ShapeDtypeStruct((S, E), jnp.float32),
        ],
    )(ctx, Wo, bo2, xf, n2_scale2, n2_shift2, gate_w)

    out = pl.pallas_call(
        _moe_dense_kernel,
        grid=(E, NHC),
        in_specs=[
            pl.BlockSpec((S, D), lambda e, hc: (0, 0)),
            pl.BlockSpec((S, D), lambda e, hc: (0, 0)),
            pl.BlockSpec((S, E), lambda e, hc: (0, 0)),
            pl.BlockSpec((1, D, HC), lambda e, hc: (e, 0, hc)),
            pl.BlockSpec((1, D, HC), lambda e, hc: (e, 0, hc)),
            pl.BlockSpec((1, HC, D), lambda e, hc: (e, hc, 0)),
        ],
        out_specs=pl.BlockSpec((S, D), lambda e, hc: (0, 0)),
        out_shape=jax.ShapeDtypeStruct((S, D), jnp.float32),
    )(h2, x2, wf, fc1_w, fc2_w, fc3_w)

    return out.reshape(b, s, d)
